# Initial kernel scaffold; baseline (speedup 1.0000x reference)
import functools

import jax
import jax.numpy as jnp
from jax import lax
from jax.experimental import pallas as pl
from jax.experimental.pallas import tpu as pltpu
from jax.experimental.pallas import tpu_sc as plsc

NC = 2
NS = 16
NW = NC * NS
CHUNK = 128

_MESH = plsc.VectorSubcoreMesh(
    core_axis_name="c", subcore_axis_name="s", num_cores=NC, num_subcores=NS
)


def _make_deg_kernel(n_chunks, npad):
    rows_per_sub = npad // NS

    @functools.partial(
        pl.kernel,
        out_type=jax.ShapeDtypeStruct((NC, npad, 16), jnp.float32),
        mesh=_MESH,
        scratch_types=[
            pltpu.VMEM((n_chunks, CHUNK), jnp.int32),
            pltpu.VMEM((CHUNK, 16), jnp.float32),
            pltpu.VMEM_SHARED((npad, 16), jnp.float32),
        ],
    )
    def deg_kernel(dst_hbm, zeros_hbm, ones_hbm, out_hbm, dst_v, ones_v, acc_sh):
        cid = lax.axis_index("c")
        sid = lax.axis_index("s")
        wid = sid * NC + cid
        pltpu.sync_copy(
            zeros_hbm.at[pl.ds(sid * rows_per_sub, rows_per_sub)],
            acc_sh.at[pl.ds(sid * rows_per_sub, rows_per_sub)],
        )
        pltpu.sync_copy(ones_hbm, ones_v)
        pltpu.sync_copy(dst_hbm.at[pl.ds(wid * n_chunks, n_chunks)], dst_v)
        plsc.subcore_barrier()

        @pl.loop(0, n_chunks)
        def _(c):
            pltpu.sync_copy(ones_v, acc_sh.at[dst_v.at[c]], add=True)

        plsc.subcore_barrier()
        pltpu.sync_copy(
            acc_sh.at[pl.ds(sid * rows_per_sub, rows_per_sub)],
            out_hbm.at[cid, pl.ds(sid * rows_per_sub, rows_per_sub)],
        )

    return deg_kernel


def _make_edge_kernel(n_chunks, npad, dim):
    rows_per_sub = npad // NS

    @functools.partial(
        pl.kernel,
        out_type=jax.ShapeDtypeStruct((NC, npad, dim), jnp.float32),
        mesh=_MESH,
        scratch_types=[
            pltpu.VMEM((n_chunks, CHUNK), jnp.int32),
            pltpu.VMEM((n_chunks, CHUNK), jnp.int32),
            pltpu.VMEM((CHUNK, dim), jnp.float32),
            pltpu.VMEM_SHARED((npad, dim), jnp.float32),
            pltpu.SemaphoreType.DMA,
        ],
    )
    def edge_kernel(
        src_hbm, dst_hbm, hp_hbm, zeros_hbm, out_hbm,
        src_v, dst_v, rows_v, acc_sh, sem,
    ):
        cid = lax.axis_index("c")
        sid = lax.axis_index("s")
        wid = sid * NC + cid
        pltpu.sync_copy(
            zeros_hbm.at[pl.ds(sid * rows_per_sub, rows_per_sub)],
            acc_sh.at[pl.ds(sid * rows_per_sub, rows_per_sub)],
        )
        pltpu.sync_copy(src_hbm.at[pl.ds(wid * n_chunks, n_chunks)], src_v)
        pltpu.sync_copy(dst_hbm.at[pl.ds(wid * n_chunks, n_chunks)], dst_v)
        plsc.subcore_barrier()

        @pl.loop(0, n_chunks)
        def _(c):
            pltpu.async_copy(hp_hbm.at[src_v.at[c]], rows_v, sem).wait()
            pltpu.sync_copy(rows_v, acc_sh.at[dst_v.at[c]], add=True)

        plsc.subcore_barrier()
        pltpu.sync_copy(
            acc_sh.at[pl.ds(sid * rows_per_sub, rows_per_sub)],
            out_hbm.at[cid, pl.ds(sid * rows_per_sub, rows_per_sub)],
        )

    return edge_kernel


def _mm1_body(h0_ref, h1_ref, x_ref, w_ref, out_ref):
    deg = h0_ref[...][:, 0:1] + h1_ref[...][:, 0:1] + 1.0
    dinv = lax.rsqrt(deg)
    h = jnp.dot(x_ref[...], w_ref[...], preferred_element_type=jnp.float32)
    out_ref[...] = h * dinv


def _final_body(a0_ref, a1_ref, hp_ref, h0_ref, h1_ref, b1_ref, w2_ref,
                b2_ref, out_ref):
    deg = h0_ref[...][:, 0:1] + h1_ref[...][:, 0:1] + 1.0
    dinv = lax.rsqrt(deg)
    tot = (a0_ref[...] + a1_ref[...] + hp_ref[...]) * dinv + b1_ref[...]
    r = jnp.maximum(tot, 0.0)
    z = jnp.dot(r, w2_ref[...], preferred_element_type=jnp.float32)
    z = z + b2_ref[...]
    m = jnp.max(z, axis=1, keepdims=True)
    zm = z - m
    out_ref[...] = zm - jnp.log(jnp.sum(jnp.exp(zm), axis=1, keepdims=True))


def kernel(x, edge_index, W1, b1, W2, b2):
    n, in_dim = x.shape
    hid = W1.shape[1]
    out_dim = W2.shape[1]
    e = edge_index.shape[1]

    npad = ((n + 1 + 2047) // 2048) * 2048
    dummy = npad - 1
    n_chunks = -(-e // (NW * CHUNK))
    e_pad = NW * n_chunks * CHUNK

    src = edge_index[0].astype(jnp.int32)
    dst = edge_index[1].astype(jnp.int32)
    fill = jnp.full((e_pad - e,), dummy, dtype=jnp.int32)
    src2d = jnp.concatenate([src, fill]).reshape(NW * n_chunks, CHUNK)
    dst2d = jnp.concatenate([dst, fill]).reshape(NW * n_chunks, CHUNK)

    zeros16 = jnp.zeros((npad, 16), jnp.float32)
    ones16 = jnp.ones((CHUNK, 16), jnp.float32)
    zerosd = jnp.zeros((npad, hid), jnp.float32)
    x_pad = jnp.pad(x, ((0, npad - n), (0, 0)))

    hist2 = _make_deg_kernel(n_chunks, npad)(dst2d, zeros16, ones16)

    blk1 = 2048
    hp = pl.pallas_call(
        _mm1_body,
        grid=(npad // blk1,),
        in_specs=[
            pl.BlockSpec((blk1, 16), lambda i: (i, 0)),
            pl.BlockSpec((blk1, 16), lambda i: (i, 0)),
            pl.BlockSpec((blk1, in_dim), lambda i: (i, 0)),
            pl.BlockSpec((in_dim, hid), lambda i: (0, 0)),
        ],
        out_specs=pl.BlockSpec((blk1, hid), lambda i: (i, 0)),
        out_shape=jax.ShapeDtypeStruct((npad, hid), jnp.float32),
    )(hist2[0], hist2[1], x_pad, W1)

    acc2 = _make_edge_kernel(n_chunks, npad, hid)(src2d, dst2d, hp, zerosd)

    blk2 = 2000
    out = pl.pallas_call(
        _final_body,
        grid=(n // blk2,),
        in_specs=[
            pl.BlockSpec((blk2, hid), lambda i: (i, 0)),
            pl.BlockSpec((blk2, hid), lambda i: (i, 0)),
            pl.BlockSpec((blk2, hid), lambda i: (i, 0)),
            pl.BlockSpec((blk2, 16), lambda i: (i, 0)),
            pl.BlockSpec((blk2, 16), lambda i: (i, 0)),
            pl.BlockSpec((1, hid), lambda i: (0, 0)),
            pl.BlockSpec((hid, out_dim), lambda i: (0, 0)),
            pl.BlockSpec((1, out_dim), lambda i: (0, 0)),
        ],
        out_specs=pl.BlockSpec((blk2, out_dim), lambda i: (i, 0)),
        out_shape=jax.ShapeDtypeStruct((n, out_dim), jnp.float32),
    )(acc2[0], acc2[1], hp, hist2[0], hist2[1],
      b1.reshape(1, hid), W2, b2.reshape(1, out_dim))

    return out


# speedup vs baseline: 11.2974x; 11.2974x over previous
import functools

import jax
import jax.numpy as jnp
from jax import lax
from jax.experimental import pallas as pl
from jax.experimental.pallas import tpu as pltpu
from jax.experimental.pallas import tpu_sc as plsc

NC = 2
NS = 16
NW = NC * NS
CHUNK = 128

def _mesh():
    return plsc.VectorSubcoreMesh(
        core_axis_name="c", subcore_axis_name="s", num_cores=NC, num_subcores=NS
    )


def _make_deg_kernel(n_chunks, npad):
    rows_per_sub = npad // NS

    @functools.partial(
        pl.kernel,
        out_type=jax.ShapeDtypeStruct((NC, npad, 16), jnp.float32),
        mesh=_mesh(),
        scratch_types=[
            pltpu.VMEM((n_chunks, CHUNK), jnp.int32),
            pltpu.VMEM((CHUNK, 16), jnp.float32),
            pltpu.VMEM_SHARED((npad, 16), jnp.float32),
        ],
        compiler_params=pltpu.CompilerParams(use_tc_tiling_on_sc=False),
    )
    def deg_kernel(dst_hbm, zeros_hbm, ones_hbm, out_hbm, dst_v, ones_v, acc_sh):
        cid = lax.axis_index("c")
        sid = lax.axis_index("s")
        wid = sid * NC + cid
        pltpu.sync_copy(
            zeros_hbm.at[pl.ds(sid * rows_per_sub, rows_per_sub)],
            acc_sh.at[pl.ds(sid * rows_per_sub, rows_per_sub)],
        )
        pltpu.sync_copy(ones_hbm, ones_v)
        pltpu.sync_copy(dst_hbm.at[pl.ds(wid * n_chunks, n_chunks)], dst_v)
        plsc.subcore_barrier()

        @pl.loop(0, n_chunks)
        def _(c):
            pltpu.sync_copy(ones_v, acc_sh.at[dst_v.at[c]], add=True)

        plsc.subcore_barrier()
        pltpu.sync_copy(
            acc_sh.at[pl.ds(sid * rows_per_sub, rows_per_sub)],
            out_hbm.at[cid, pl.ds(sid * rows_per_sub, rows_per_sub)],
        )

    return deg_kernel


def _make_edge_kernel(n_chunks, npad, dim):
    rows_per_sub = npad // NS

    @functools.partial(
        pl.kernel,
        out_type=jax.ShapeDtypeStruct((NC, npad, dim), jnp.float32),
        mesh=_mesh(),
        scratch_types=[
            pltpu.VMEM((n_chunks, CHUNK), jnp.int32),
            pltpu.VMEM((n_chunks, CHUNK), jnp.int32),
            pltpu.VMEM((CHUNK, dim), jnp.float32),
            pltpu.VMEM_SHARED((npad, dim), jnp.float32),
            pltpu.SemaphoreType.DMA,
        ],
    )
    def edge_kernel(
        src_hbm, dst_hbm, hp_hbm, zeros_hbm, out_hbm,
        src_v, dst_v, rows_v, acc_sh, sem,
    ):
        cid = lax.axis_index("c")
        sid = lax.axis_index("s")
        wid = sid * NC + cid
        pltpu.sync_copy(
            zeros_hbm.at[pl.ds(sid * rows_per_sub, rows_per_sub)],
            acc_sh.at[pl.ds(sid * rows_per_sub, rows_per_sub)],
        )
        pltpu.sync_copy(src_hbm.at[pl.ds(wid * n_chunks, n_chunks)], src_v)
        pltpu.sync_copy(dst_hbm.at[pl.ds(wid * n_chunks, n_chunks)], dst_v)
        plsc.subcore_barrier()

        @pl.loop(0, n_chunks)
        def _(c):
            pltpu.async_copy(hp_hbm.at[src_v.at[c]], rows_v, sem).wait()
            pltpu.sync_copy(rows_v, acc_sh.at[dst_v.at[c]], add=True)

        plsc.subcore_barrier()
        pltpu.sync_copy(
            acc_sh.at[pl.ds(sid * rows_per_sub, rows_per_sub)],
            out_hbm.at[cid, pl.ds(sid * rows_per_sub, rows_per_sub)],
        )

    return edge_kernel


def _mm1_body(h0_ref, h1_ref, x_ref, w_ref, out_ref):
    deg = h0_ref[...][:, 0:1] + h1_ref[...][:, 0:1] + 1.0
    dinv = lax.rsqrt(deg)
    h = jnp.dot(x_ref[...], w_ref[...], preferred_element_type=jnp.float32)
    out_ref[...] = h * dinv


def _final_body(a0_ref, a1_ref, hp_ref, h0_ref, h1_ref, b1_ref, w2_ref,
                b2_ref, out_ref):
    deg = h0_ref[...][:, 0:1] + h1_ref[...][:, 0:1] + 1.0
    dinv = lax.rsqrt(deg)
    tot = (a0_ref[...] + a1_ref[...] + hp_ref[...]) * dinv + b1_ref[...]
    r = jnp.maximum(tot, 0.0)
    z = jnp.dot(r, w2_ref[...], preferred_element_type=jnp.float32)
    z = z + b2_ref[...]
    m = jnp.max(z, axis=1, keepdims=True)
    zm = z - m
    out_ref[...] = zm - jnp.log(jnp.sum(jnp.exp(zm), axis=1, keepdims=True))


def kernel(x, edge_index, W1, b1, W2, b2):
    n, in_dim = x.shape
    hid = W1.shape[1]
    out_dim = W2.shape[1]
    e = edge_index.shape[1]

    npad = ((n + 1 + 2047) // 2048) * 2048
    dummy = npad - 1
    n_chunks = ((-(-e // (NW * CHUNK)) + 7) // 8) * 8
    e_pad = NW * n_chunks * CHUNK

    src = edge_index[0].astype(jnp.int32)
    dst = edge_index[1].astype(jnp.int32)
    fill = jnp.full((e_pad - e,), dummy, dtype=jnp.int32)
    src2d = jnp.concatenate([src, fill]).reshape(NW * n_chunks, CHUNK)
    dst2d = jnp.concatenate([dst, fill]).reshape(NW * n_chunks, CHUNK)

    zeros16 = jnp.zeros((npad, 16), jnp.float32)
    ones16 = jnp.ones((CHUNK, 16), jnp.float32)
    zerosd = jnp.zeros((npad, hid), jnp.float32)
    x_pad = jnp.pad(x, ((0, npad - n), (0, 0)))

    hist2 = _make_deg_kernel(n_chunks, npad)(dst2d, zeros16, ones16)

    blk1 = 2048
    hp = pl.pallas_call(
        _mm1_body,
        grid=(npad // blk1,),
        in_specs=[
            pl.BlockSpec((blk1, 16), lambda i: (i, 0)),
            pl.BlockSpec((blk1, 16), lambda i: (i, 0)),
            pl.BlockSpec((blk1, in_dim), lambda i: (i, 0)),
            pl.BlockSpec((in_dim, hid), lambda i: (0, 0)),
        ],
        out_specs=pl.BlockSpec((blk1, hid), lambda i: (i, 0)),
        out_shape=jax.ShapeDtypeStruct((npad, hid), jnp.float32),
    )(hist2[0], hist2[1], x_pad, W1)

    acc2 = _make_edge_kernel(n_chunks, npad, hid)(src2d, dst2d, hp, zerosd)

    blk2 = 2000
    out = pl.pallas_call(
        _final_body,
        grid=(n // blk2,),
        in_specs=[
            pl.BlockSpec((blk2, hid), lambda i: (i, 0)),
            pl.BlockSpec((blk2, hid), lambda i: (i, 0)),
            pl.BlockSpec((blk2, hid), lambda i: (i, 0)),
            pl.BlockSpec((blk2, 16), lambda i: (i, 0)),
            pl.BlockSpec((blk2, 16), lambda i: (i, 0)),
            pl.BlockSpec((1, hid), lambda i: (0, 0)),
            pl.BlockSpec((hid, out_dim), lambda i: (0, 0)),
            pl.BlockSpec((1, out_dim), lambda i: (0, 0)),
        ],
        out_specs=pl.BlockSpec((blk2, out_dim), lambda i: (i, 0)),
        out_shape=jax.ShapeDtypeStruct((n, out_dim), jnp.float32),
    )(acc2[0], acc2[1], hp, hist2[0], hist2[1],
      b1.reshape(1, hid), W2, b2.reshape(1, out_dim))

    return out


# speedup vs baseline: 32.8706x; 2.9096x over previous
import functools

import jax
import jax.numpy as jnp
from jax import lax
from jax.experimental import pallas as pl
from jax.experimental.pallas import tpu as pltpu
from jax.experimental.pallas import tpu_sc as plsc

NC = 2
NS = 16
NW = NC * NS
CHUNK = 128

def _mesh():
    return plsc.VectorSubcoreMesh(
        core_axis_name="c", subcore_axis_name="s", num_cores=NC, num_subcores=NS
    )


def _make_deg_kernel(n_chunks, npad):
    rows_per_sub = npad // NS

    @functools.partial(
        pl.kernel,
        out_type=jax.ShapeDtypeStruct((NC, npad, 16), jnp.float32),
        mesh=_mesh(),
        scratch_types=[
            pltpu.VMEM((n_chunks, CHUNK), jnp.int32),
            pltpu.VMEM((CHUNK, 16), jnp.float32),
            pltpu.VMEM_SHARED((npad, 16), jnp.float32),
        ],
        compiler_params=pltpu.CompilerParams(use_tc_tiling_on_sc=False),
    )
    def deg_kernel(dst_hbm, zeros_hbm, ones_hbm, out_hbm, dst_v, ones_v, acc_sh):
        cid = lax.axis_index("c")
        sid = lax.axis_index("s")
        wid = sid * NC + cid
        pltpu.sync_copy(
            zeros_hbm.at[pl.ds(sid * rows_per_sub, rows_per_sub)],
            acc_sh.at[pl.ds(sid * rows_per_sub, rows_per_sub)],
        )
        pltpu.sync_copy(ones_hbm, ones_v)
        pltpu.sync_copy(dst_hbm.at[pl.ds(wid * n_chunks, n_chunks)], dst_v)
        plsc.subcore_barrier()

        @pl.loop(0, n_chunks)
        def _(c):
            pltpu.sync_copy(ones_v, acc_sh.at[dst_v.at[c]], add=True)

        plsc.subcore_barrier()
        pltpu.sync_copy(
            acc_sh.at[pl.ds(sid * rows_per_sub, rows_per_sub)],
            out_hbm.at[cid, pl.ds(sid * rows_per_sub, rows_per_sub)],
        )

    return deg_kernel


def _make_edge_kernel(n_chunks, npad, dim):
    rows_per_sub = npad // NS
    half = n_chunks // 2

    @functools.partial(
        pl.kernel,
        out_type=jax.ShapeDtypeStruct((NC, npad, dim), jnp.float32),
        mesh=_mesh(),
        scratch_types=[
            pltpu.VMEM((half, CHUNK), jnp.int32),
            pltpu.VMEM((half, CHUNK), jnp.int32),
            pltpu.VMEM((CHUNK, dim), jnp.float32),
            pltpu.VMEM((CHUNK, dim), jnp.float32),
            pltpu.VMEM_SHARED((npad, dim), jnp.float32),
            pltpu.SemaphoreType.DMA,
            pltpu.SemaphoreType.DMA,
        ],
    )
    def edge_kernel(
        src_hbm, dst_hbm, hp_hbm, zeros_hbm, out_hbm,
        src_v, dst_v, rows0_v, rows1_v, acc_sh, sem0, sem1,
    ):
        cid = lax.axis_index("c")
        sid = lax.axis_index("s")
        wid = sid * NC + cid
        pltpu.sync_copy(
            zeros_hbm.at[pl.ds(sid * rows_per_sub, rows_per_sub)],
            acc_sh.at[pl.ds(sid * rows_per_sub, rows_per_sub)],
        )
        plsc.subcore_barrier()

        for h in range(2):
            base = wid * n_chunks + h * half
            pltpu.sync_copy(src_hbm.at[pl.ds(base, half)], src_v)
            pltpu.sync_copy(dst_hbm.at[pl.ds(base, half)], dst_v)

            pltpu.async_copy(hp_hbm.at[src_v.at[0]], rows0_v, sem0)

            @pl.loop(0, half // 2)
            def _(i):
                c0 = 2 * i
                pltpu.make_async_copy(
                    hp_hbm.at[src_v.at[c0]], rows0_v, sem0
                ).wait()
                pltpu.async_copy(hp_hbm.at[src_v.at[c0 + 1]], rows1_v, sem1)
                pltpu.sync_copy(rows0_v, acc_sh.at[dst_v.at[c0]], add=True)
                pltpu.make_async_copy(
                    hp_hbm.at[src_v.at[c0 + 1]], rows1_v, sem1
                ).wait()

                @pl.when(c0 + 2 < half)
                def _():
                    pltpu.async_copy(hp_hbm.at[src_v.at[c0 + 2]], rows0_v, sem0)

                pltpu.sync_copy(rows1_v, acc_sh.at[dst_v.at[c0 + 1]], add=True)

        plsc.subcore_barrier()
        pltpu.sync_copy(
            acc_sh.at[pl.ds(sid * rows_per_sub, rows_per_sub)],
            out_hbm.at[cid, pl.ds(sid * rows_per_sub, rows_per_sub)],
        )

    return edge_kernel


def _mm1_body(h0_ref, h1_ref, x_ref, w_ref, out_ref):
    deg = h0_ref[...][:, 0:1] + h1_ref[...][:, 0:1] + 1.0
    dinv = lax.rsqrt(deg)
    h = jnp.dot(x_ref[...], w_ref[...], preferred_element_type=jnp.float32)
    out_ref[...] = h * dinv


def _final_body(a0_ref, a1_ref, hp_ref, h0_ref, h1_ref, b1_ref, w2_ref,
                b2_ref, out_ref):
    deg = h0_ref[...][:, 0:1] + h1_ref[...][:, 0:1] + 1.0
    dinv = lax.rsqrt(deg)
    tot = (a0_ref[...] + a1_ref[...] + hp_ref[...]) * dinv + b1_ref[...]
    r = jnp.maximum(tot, 0.0)
    z = jnp.dot(r, w2_ref[...], preferred_element_type=jnp.float32)
    z = z + b2_ref[...]
    m = jnp.max(z, axis=1, keepdims=True)
    zm = z - m
    out_ref[...] = zm - jnp.log(jnp.sum(jnp.exp(zm), axis=1, keepdims=True))


def kernel(x, edge_index, W1, b1, W2, b2):
    n, in_dim = x.shape
    hid = W1.shape[1]
    out_dim = W2.shape[1]
    e = edge_index.shape[1]

    npad = ((n + 1 + 127) // 128) * 128
    dummy = npad - 1
    n_chunks = ((-(-e // (NW * CHUNK)) + 7) // 8) * 8
    e_pad = NW * n_chunks * CHUNK

    src = edge_index[0].astype(jnp.int32)
    dst = edge_index[1].astype(jnp.int32)
    fill = n + jax.lax.rem(
        jnp.arange(e_pad - e, dtype=jnp.int32), jnp.int32(npad - n)
    )
    src2d = jnp.concatenate([src, fill]).reshape(NW * n_chunks, CHUNK)
    dst2d = jnp.concatenate([dst, fill]).reshape(NW * n_chunks, CHUNK)

    zeros16 = jnp.zeros((npad, 16), jnp.float32)
    ones16 = jnp.ones((CHUNK, 16), jnp.float32)
    zerosd = jnp.zeros((npad, hid), jnp.float32)
    x_pad = jnp.pad(x, ((0, npad - n), (0, 0)))

    hist2 = _make_deg_kernel(n_chunks, npad)(dst2d, zeros16, ones16)

    blk1 = npad // 8
    hp = pl.pallas_call(
        _mm1_body,
        grid=(npad // blk1,),
        in_specs=[
            pl.BlockSpec((blk1, 16), lambda i: (i, 0)),
            pl.BlockSpec((blk1, 16), lambda i: (i, 0)),
            pl.BlockSpec((blk1, in_dim), lambda i: (i, 0)),
            pl.BlockSpec((in_dim, hid), lambda i: (0, 0)),
        ],
        out_specs=pl.BlockSpec((blk1, hid), lambda i: (i, 0)),
        out_shape=jax.ShapeDtypeStruct((npad, hid), jnp.float32),
    )(hist2[0], hist2[1], x_pad, W1)

    acc2 = _make_edge_kernel(n_chunks, npad, hid)(src2d, dst2d, hp, zerosd)

    blk2 = 2000
    out = pl.pallas_call(
        _final_body,
        grid=(n // blk2,),
        in_specs=[
            pl.BlockSpec((blk2, hid), lambda i: (i, 0)),
            pl.BlockSpec((blk2, hid), lambda i: (i, 0)),
            pl.BlockSpec((blk2, hid), lambda i: (i, 0)),
            pl.BlockSpec((blk2, 16), lambda i: (i, 0)),
            pl.BlockSpec((blk2, 16), lambda i: (i, 0)),
            pl.BlockSpec((1, hid), lambda i: (0, 0)),
            pl.BlockSpec((hid, out_dim), lambda i: (0, 0)),
            pl.BlockSpec((1, out_dim), lambda i: (0, 0)),
        ],
        out_specs=pl.BlockSpec((blk2, out_dim), lambda i: (i, 0)),
        out_shape=jax.ShapeDtypeStruct((n, out_dim), jnp.float32),
    )(acc2[0], acc2[1], hp, hist2[0], hist2[1],
      b1.reshape(1, hid), W2, b2.reshape(1, out_dim))

    return out


# speedup vs baseline: 35.7239x; 1.0868x over previous
import functools

import jax
import jax.numpy as jnp
from jax import lax
from jax.experimental import pallas as pl
from jax.experimental.pallas import tpu as pltpu
from jax.experimental.pallas import tpu_sc as plsc

NC = 2
NS = 16
NW = NC * NS
NBUF = 4
CHUNK = 64

def _mesh():
    return plsc.VectorSubcoreMesh(
        core_axis_name="c", subcore_axis_name="s", num_cores=NC, num_subcores=NS
    )


def _make_deg_kernel(n_chunks, npad):
    rows_per_sub = npad // NS

    @functools.partial(
        pl.kernel,
        out_type=jax.ShapeDtypeStruct((NC, npad, 16), jnp.float32),
        mesh=_mesh(),
        scratch_types=[
            pltpu.VMEM((n_chunks, CHUNK), jnp.int32),
            pltpu.VMEM((CHUNK, 16), jnp.float32),
            pltpu.VMEM_SHARED((npad, 16), jnp.float32),
        ],
        compiler_params=pltpu.CompilerParams(use_tc_tiling_on_sc=False),
    )
    def deg_kernel(dst_hbm, zeros_hbm, ones_hbm, out_hbm, dst_v, ones_v, acc_sh):
        cid = lax.axis_index("c")
        sid = lax.axis_index("s")
        wid = sid * NC + cid
        pltpu.sync_copy(
            zeros_hbm.at[pl.ds(sid * rows_per_sub, rows_per_sub)],
            acc_sh.at[pl.ds(sid * rows_per_sub, rows_per_sub)],
        )
        pltpu.sync_copy(ones_hbm, ones_v)
        pltpu.sync_copy(dst_hbm.at[pl.ds(wid * n_chunks, n_chunks)], dst_v)
        plsc.subcore_barrier()

        @pl.loop(0, n_chunks)
        def _(c):
            pltpu.sync_copy(ones_v, acc_sh.at[dst_v.at[c]], add=True)

        plsc.subcore_barrier()
        pltpu.sync_copy(
            acc_sh.at[pl.ds(sid * rows_per_sub, rows_per_sub)],
            out_hbm.at[cid, pl.ds(sid * rows_per_sub, rows_per_sub)],
        )

    return deg_kernel


def _make_edge_kernel(n_chunks, npad, dim):
    rows_per_sub = npad // NS
    half = n_chunks // 4

    @functools.partial(
        pl.kernel,
        out_type=jax.ShapeDtypeStruct((NC, npad, dim), jnp.float32),
        mesh=_mesh(),
        scratch_types=(
            [
                pltpu.VMEM((half, CHUNK), jnp.int32),
                pltpu.VMEM((half, CHUNK), jnp.int32),
            ]
            + [pltpu.VMEM((CHUNK, dim), jnp.float32)] * NBUF
            + [pltpu.VMEM_SHARED((npad, dim), jnp.float32)]
            + [pltpu.SemaphoreType.DMA] * (2 * NBUF)
        ),
    )
    def edge_kernel(src_hbm, dst_hbm, hp_hbm, zeros_hbm, out_hbm, *scr):
        src_v, dst_v = scr[0], scr[1]
        rows = scr[2:2 + NBUF]
        acc_sh = scr[2 + NBUF]
        gsem = scr[3 + NBUF:3 + 2 * NBUF]
        ssem = scr[3 + 2 * NBUF:3 + 3 * NBUF]
        cid = lax.axis_index("c")
        sid = lax.axis_index("s")
        wid = sid * NC + cid
        pltpu.sync_copy(
            zeros_hbm.at[pl.ds(sid * rows_per_sub, rows_per_sub)],
            acc_sh.at[pl.ds(sid * rows_per_sub, rows_per_sub)],
        )
        plsc.subcore_barrier()

        for h in range(4):
            base = wid * n_chunks + h * half
            pltpu.sync_copy(src_hbm.at[pl.ds(base, half)], src_v)
            pltpu.sync_copy(dst_hbm.at[pl.ds(base, half)], dst_v)

            for k in range(NBUF):
                pltpu.async_copy(hp_hbm.at[src_v.at[k]], rows[k], gsem[k])

            @pl.loop(0, half // NBUF)
            def _(i):
                c = i * NBUF
                for k in range(NBUF):
                    pltpu.make_async_copy(
                        hp_hbm.at[src_v.at[c + k]], rows[k], gsem[k]
                    ).wait()
                    pltpu.async_copy(
                        rows[k], acc_sh.at[dst_v.at[c + k]], ssem[k], add=True
                    )
                for k in range(NBUF):
                    pltpu.make_async_copy(
                        rows[k], acc_sh.at[dst_v.at[c + k]], ssem[k]
                    ).wait()

                    @pl.when(c + NBUF + k < half)
                    def _():
                        pltpu.async_copy(
                            hp_hbm.at[src_v.at[c + NBUF + k]], rows[k], gsem[k]
                        )

        plsc.subcore_barrier()
        pltpu.sync_copy(
            acc_sh.at[pl.ds(sid * rows_per_sub, rows_per_sub)],
            out_hbm.at[cid, pl.ds(sid * rows_per_sub, rows_per_sub)],
        )

    return edge_kernel


def _mm1_body(h0_ref, h1_ref, x_ref, w_ref, out_ref):
    deg = h0_ref[0][:, 0:1] + h1_ref[0][:, 0:1] + 1.0
    dinv = lax.rsqrt(deg)
    h = jnp.dot(x_ref[...], w_ref[...], preferred_element_type=jnp.float32)
    out_ref[...] = h * dinv


def _final_body(a0_ref, a1_ref, hp_ref, h0_ref, h1_ref, b1_ref, w2_ref,
                b2_ref, out_ref):
    deg = h0_ref[0][:, 0:1] + h1_ref[0][:, 0:1] + 1.0
    dinv = lax.rsqrt(deg)
    tot = (a0_ref[0] + a1_ref[0] + hp_ref[...]) * dinv + b1_ref[...]
    r = jnp.maximum(tot, 0.0)
    z = jnp.dot(r, w2_ref[...], preferred_element_type=jnp.float32)
    z = z + b2_ref[...]
    m = jnp.max(z, axis=1, keepdims=True)
    zm = z - m
    out_ref[...] = zm - jnp.log(jnp.sum(jnp.exp(zm), axis=1, keepdims=True))


def kernel(x, edge_index, W1, b1, W2, b2):
    n, in_dim = x.shape
    hid = W1.shape[1]
    out_dim = W2.shape[1]
    e = edge_index.shape[1]

    npad = ((n + 1 + 127) // 128) * 128
    dummy = npad - 1
    n_chunks = ((-(-e // (NW * CHUNK)) + 7) // 8) * 8
    e_pad = NW * n_chunks * CHUNK

    src = edge_index[0].astype(jnp.int32)
    dst = edge_index[1].astype(jnp.int32)
    fill = n + jax.lax.rem(
        jnp.arange(e_pad - e, dtype=jnp.int32), jnp.int32(npad - n)
    )
    src2d = jnp.concatenate([src, fill]).reshape(NW * n_chunks, CHUNK)
    dst2d = jnp.concatenate([dst, fill]).reshape(NW * n_chunks, CHUNK)

    zeros16 = jnp.zeros((npad, 16), jnp.float32)
    ones16 = jnp.ones((CHUNK, 16), jnp.float32)
    zerosd = jnp.zeros((npad, hid), jnp.float32)
    x_pad = jnp.pad(x, ((0, npad - n), (0, 0)))

    hist2 = _make_deg_kernel(n_chunks, npad)(dst2d, zeros16, ones16)

    blk1 = npad // 8
    hp = pl.pallas_call(
        _mm1_body,
        grid=(npad // blk1,),
        in_specs=[
            pl.BlockSpec((1, blk1, 16), lambda i: (0, i, 0)),
            pl.BlockSpec((1, blk1, 16), lambda i: (1, i, 0)),
            pl.BlockSpec((blk1, in_dim), lambda i: (i, 0)),
            pl.BlockSpec((in_dim, hid), lambda i: (0, 0)),
        ],
        out_specs=pl.BlockSpec((blk1, hid), lambda i: (i, 0)),
        out_shape=jax.ShapeDtypeStruct((npad, hid), jnp.float32),
    )(hist2, hist2, x_pad, W1)

    acc2 = _make_edge_kernel(n_chunks, npad, hid)(src2d, dst2d, hp, zerosd)

    blk2 = 2000
    out = pl.pallas_call(
        _final_body,
        grid=(n // blk2,),
        in_specs=[
            pl.BlockSpec((1, blk2, hid), lambda i: (0, i, 0)),
            pl.BlockSpec((1, blk2, hid), lambda i: (1, i, 0)),
            pl.BlockSpec((blk2, hid), lambda i: (i, 0)),
            pl.BlockSpec((1, blk2, 16), lambda i: (0, i, 0)),
            pl.BlockSpec((1, blk2, 16), lambda i: (1, i, 0)),
            pl.BlockSpec((1, hid), lambda i: (0, 0)),
            pl.BlockSpec((hid, out_dim), lambda i: (0, 0)),
            pl.BlockSpec((1, out_dim), lambda i: (0, 0)),
        ],
        out_specs=pl.BlockSpec((blk2, out_dim), lambda i: (i, 0)),
        out_shape=jax.ShapeDtypeStruct((n, out_dim), jnp.float32),
    )(acc2, acc2, hp, hist2, hist2,
      b1.reshape(1, hid), W2, b2.reshape(1, out_dim))

    return out


# speedup vs baseline: 36.5891x; 1.0242x over previous
import functools

import jax
import jax.numpy as jnp
from jax import lax
from jax.experimental import pallas as pl
from jax.experimental.pallas import tpu as pltpu
from jax.experimental.pallas import tpu_sc as plsc

NC = 2
NS = 16
NW = NC * NS
CHUNK = 64
NBUF = 4
DEG_AHEAD = 4


def _mesh():
    return plsc.VectorSubcoreMesh(
        core_axis_name="c", subcore_axis_name="s", num_cores=NC, num_subcores=NS
    )


def _load_idx(main_hbm, fill_hbm, buf, start, q, main_rows):
    is_fill = start >= main_rows

    @pl.when(is_fill)
    def _():
        pltpu.sync_copy(fill_hbm.at[pl.ds(start - main_rows, q)], buf)

    @pl.when(jnp.logical_not(is_fill))
    def _():
        pltpu.sync_copy(main_hbm.at[pl.ds(start, q)], buf)


def _make_deg_kernel(n_chunks, npad, main_rows):
    rows_per_sub = npad // NS
    q = n_chunks // 4

    @functools.partial(
        pl.kernel,
        out_type=jax.ShapeDtypeStruct((NC, npad, 16), jnp.float32),
        mesh=_mesh(),
        scratch_types=[
            pltpu.VMEM((q, CHUNK), jnp.int32),
            pltpu.VMEM((CHUNK, 16), jnp.float32),
            pltpu.VMEM_SHARED((npad, 16), jnp.float32),
            pltpu.SemaphoreType.DMA,
        ],
        compiler_params=pltpu.CompilerParams(use_tc_tiling_on_sc=False),
    )
    def deg_kernel(dst_hbm, fill_hbm, zeros_hbm, ones_hbm, out_hbm,
                   dst_v, ones_v, acc_sh, dsem):
        cid = lax.axis_index("c")
        sid = lax.axis_index("s")
        wid = sid * NC + cid
        pltpu.sync_copy(
            zeros_hbm.at[pl.ds(sid * rows_per_sub, rows_per_sub)],
            acc_sh.at[pl.ds(sid * rows_per_sub, rows_per_sub)],
        )
        pltpu.sync_copy(ones_hbm, ones_v)
        plsc.subcore_barrier()

        for h in range(4):
            _load_idx(dst_hbm, fill_hbm, dst_v, wid * n_chunks + h * q, q,
                      main_rows)
            for k in range(DEG_AHEAD):
                pltpu.async_copy(ones_v, acc_sh.at[dst_v.at[k]], dsem,
                                 add=True)

            @pl.loop(0, q - DEG_AHEAD)
            def _(c):
                pltpu.make_async_copy(ones_v, acc_sh.at[dst_v.at[0]],
                                      dsem).wait()
                pltpu.async_copy(ones_v, acc_sh.at[dst_v.at[c + DEG_AHEAD]],
                                 dsem, add=True)

            @pl.loop(0, DEG_AHEAD)
            def _(c):
                pltpu.make_async_copy(ones_v, acc_sh.at[dst_v.at[0]],
                                      dsem).wait()

        plsc.subcore_barrier()
        pltpu.sync_copy(
            acc_sh.at[pl.ds(sid * rows_per_sub, rows_per_sub)],
            out_hbm.at[cid, pl.ds(sid * rows_per_sub, rows_per_sub)],
        )

    return deg_kernel


def _make_edge_kernel(n_chunks, npad, dim, main_rows):
    rows_per_sub = npad // NS
    q = n_chunks // 4

    @functools.partial(
        pl.kernel,
        out_type=jax.ShapeDtypeStruct((NC, npad, dim), jnp.float32),
        mesh=_mesh(),
        scratch_types=(
            [
                pltpu.VMEM((q, CHUNK), jnp.int32),
                pltpu.VMEM((q, CHUNK), jnp.int32),
            ]
            + [pltpu.VMEM((CHUNK, dim), jnp.float32)] * NBUF
            + [pltpu.VMEM_SHARED((npad, dim), jnp.float32)]
            + [pltpu.SemaphoreType.DMA] * (2 * NBUF)
        ),
    )
    def edge_kernel(src_hbm, dst_hbm, fill_hbm, hp_hbm, zeros_hbm, out_hbm,
                    *scr):
        src_v, dst_v = scr[0], scr[1]
        rows = scr[2:2 + NBUF]
        acc_sh = scr[2 + NBUF]
        gsem = scr[3 + NBUF:3 + 2 * NBUF]
        ssem = scr[3 + 2 * NBUF:3 + 3 * NBUF]
        cid = lax.axis_index("c")
        sid = lax.axis_index("s")
        wid = sid * NC + cid
        pltpu.sync_copy(
            zeros_hbm.at[pl.ds(sid * rows_per_sub, rows_per_sub)],
            acc_sh.at[pl.ds(sid * rows_per_sub, rows_per_sub)],
        )
        plsc.subcore_barrier()

        for h in range(4):
            base = wid * n_chunks + h * q
            _load_idx(src_hbm, fill_hbm, src_v, base, q, main_rows)
            _load_idx(dst_hbm, fill_hbm, dst_v, base, q, main_rows)

            for k in range(NBUF):
                pltpu.async_copy(hp_hbm.at[src_v.at[k]], rows[k], gsem[k])

            @pl.loop(0, q // NBUF)
            def _(i):
                c = i * NBUF
                for k in range(NBUF):
                    pltpu.make_async_copy(
                        hp_hbm.at[src_v.at[c + k]], rows[k], gsem[k]
                    ).wait()
                    pltpu.async_copy(
                        rows[k], acc_sh.at[dst_v.at[c + k]], ssem[k], add=True
                    )
                for k in range(NBUF):
                    pltpu.make_async_copy(
                        rows[k], acc_sh.at[dst_v.at[c + k]], ssem[k]
                    ).wait()

                    @pl.when(c + NBUF + k < q)
                    def _():
                        pltpu.async_copy(
                            hp_hbm.at[src_v.at[c + NBUF + k]], rows[k], gsem[k]
                        )

        plsc.subcore_barrier()
        pltpu.sync_copy(
            acc_sh.at[pl.ds(sid * rows_per_sub, rows_per_sub)],
            out_hbm.at[cid, pl.ds(sid * rows_per_sub, rows_per_sub)],
        )

    return edge_kernel


def _mm_body(x_ref, w_ref, out_ref):
    out_ref[...] = jnp.dot(x_ref[...], w_ref[...],
                           preferred_element_type=jnp.float32)


def _scale_body(h0_ref, h1_ref, hraw_ref, out_ref):
    deg = h0_ref[0][:, 0:1] + h1_ref[0][:, 0:1] + 1.0
    out_ref[...] = hraw_ref[...] * lax.rsqrt(deg)


def _final_body(a0_ref, a1_ref, hp_ref, h0_ref, h1_ref, b1_ref, w2_ref,
                b2_ref, out_ref):
    deg = h0_ref[0][:, 0:1] + h1_ref[0][:, 0:1] + 1.0
    dinv = lax.rsqrt(deg)
    tot = (a0_ref[0] + a1_ref[0] + hp_ref[...]) * dinv + b1_ref[...]
    r = jnp.maximum(tot, 0.0)
    z = jnp.dot(r, w2_ref[...], preferred_element_type=jnp.float32)
    z = z + b2_ref[...]
    m = jnp.max(z, axis=1, keepdims=True)
    zm = z - m
    out_ref[...] = zm - jnp.log(jnp.sum(jnp.exp(zm), axis=1, keepdims=True))


def kernel(x, edge_index, W1, b1, W2, b2):
    n, in_dim = x.shape
    hid = W1.shape[1]
    out_dim = W2.shape[1]
    e = edge_index.shape[1]

    npad = ((n + 1 + 127) // 128) * 128
    n_chunks = ((-(-e // (NW * CHUNK)) + 7) // 8) * 8
    q = n_chunks // 4
    assert e % CHUNK == 0 and (e // CHUNK) % q == 0, (
        "edge tail must align to whole staging quarters"
    )
    main_rows = e // CHUNK
    fill_rows = NW * n_chunks - main_rows

    src2d = edge_index[0].astype(jnp.int32).reshape(main_rows, CHUNK)
    dst2d = edge_index[1].astype(jnp.int32).reshape(main_rows, CHUNK)
    fill2d = (n + jax.lax.rem(
        jnp.arange(fill_rows * CHUNK, dtype=jnp.int32), jnp.int32(npad - n)
    )).reshape(fill_rows, CHUNK)

    zeros16 = jnp.zeros((npad, 16), jnp.float32)
    ones16 = jnp.ones((CHUNK, 16), jnp.float32)
    zerosd = jnp.zeros((npad, hid), jnp.float32)
    x_pad = jnp.pad(x, ((0, npad - n), (0, 0)))

    hist2 = _make_deg_kernel(n_chunks, npad, main_rows)(
        dst2d, fill2d, zeros16, ones16)

    blk1 = npad // 8
    h_raw = pl.pallas_call(
        _mm_body,
        grid=(npad // blk1,),
        in_specs=[
            pl.BlockSpec((blk1, in_dim), lambda i: (i, 0)),
            pl.BlockSpec((in_dim, hid), lambda i: (0, 0)),
        ],
        out_specs=pl.BlockSpec((blk1, hid), lambda i: (i, 0)),
        out_shape=jax.ShapeDtypeStruct((npad, hid), jnp.float32),
    )(x_pad, W1)

    hp = pl.pallas_call(
        _scale_body,
        grid=(npad // blk1,),
        in_specs=[
            pl.BlockSpec((1, blk1, 16), lambda i: (0, i, 0)),
            pl.BlockSpec((1, blk1, 16), lambda i: (1, i, 0)),
            pl.BlockSpec((blk1, hid), lambda i: (i, 0)),
        ],
        out_specs=pl.BlockSpec((blk1, hid), lambda i: (i, 0)),
        out_shape=jax.ShapeDtypeStruct((npad, hid), jnp.float32),
    )(hist2, hist2, h_raw)

    acc2 = _make_edge_kernel(n_chunks, npad, hid, main_rows)(
        src2d, dst2d, fill2d, hp, zerosd)

    blk2 = 2000
    out = pl.pallas_call(
        _final_body,
        grid=(n // blk2,),
        in_specs=[
            pl.BlockSpec((1, blk2, hid), lambda i: (0, i, 0)),
            pl.BlockSpec((1, blk2, hid), lambda i: (1, i, 0)),
            pl.BlockSpec((blk2, hid), lambda i: (i, 0)),
            pl.BlockSpec((1, blk2, 16), lambda i: (0, i, 0)),
            pl.BlockSpec((1, blk2, 16), lambda i: (1, i, 0)),
            pl.BlockSpec((1, hid), lambda i: (0, 0)),
            pl.BlockSpec((hid, out_dim), lambda i: (0, 0)),
            pl.BlockSpec((1, out_dim), lambda i: (0, 0)),
        ],
        out_specs=pl.BlockSpec((blk2, out_dim), lambda i: (i, 0)),
        out_shape=jax.ShapeDtypeStruct((n, out_dim), jnp.float32),
    )(acc2, acc2, hp, hist2, hist2,
      b1.reshape(1, hid), W2, b2.reshape(1, out_dim))

    return out


# speedup vs baseline: 38.9344x; 1.0641x over previous
import functools

import jax
import jax.numpy as jnp
from jax import lax
from jax.experimental import pallas as pl
from jax.experimental.pallas import tpu as pltpu
from jax.experimental.pallas import tpu_sc as plsc

NC = 2
NS = 16
NW = NC * NS
CHUNK = 64
NBUF = 4
DEG_AHEAD = 4


def _mesh():
    return plsc.VectorSubcoreMesh(
        core_axis_name="c", subcore_axis_name="s", num_cores=NC, num_subcores=NS
    )


def _load_idx(idx_hbm, j, fill_hbm, buf, start, q, main_rows):
    is_fill = start >= main_rows

    @pl.when(is_fill)
    def _():
        pltpu.sync_copy(fill_hbm.at[pl.ds(start - main_rows, q)], buf)

    @pl.when(jnp.logical_not(is_fill))
    def _():
        pltpu.sync_copy(idx_hbm.at[j, pl.ds(start, q)], buf)


def _make_deg_kernel(n_chunks, npad, main_rows):
    rows_per_sub = npad // NS
    q = n_chunks // 4

    @functools.partial(
        pl.kernel,
        out_type=jax.ShapeDtypeStruct((NC, npad, 16), jnp.float32),
        mesh=_mesh(),
        scratch_types=[
            pltpu.VMEM((q, CHUNK), jnp.int32),
            pltpu.VMEM((CHUNK, 16), jnp.float32),
            pltpu.VMEM_SHARED((npad, 16), jnp.float32),
            pltpu.SemaphoreType.DMA,
        ],
        compiler_params=pltpu.CompilerParams(use_tc_tiling_on_sc=False),
    )
    def deg_kernel(idx_hbm, fill_hbm, zeros_hbm, ones_hbm, out_hbm,
                   dst_v, ones_v, acc_sh, dsem):
        cid = lax.axis_index("c")
        sid = lax.axis_index("s")
        wid = sid * NC + cid
        pltpu.sync_copy(
            zeros_hbm.at[pl.ds(sid * rows_per_sub, rows_per_sub)],
            acc_sh.at[pl.ds(sid * rows_per_sub, rows_per_sub)],
        )
        pltpu.sync_copy(ones_hbm, ones_v)
        plsc.subcore_barrier()

        for h in range(4):
            _load_idx(idx_hbm, 1, fill_hbm, dst_v, wid * n_chunks + h * q,
                      q, main_rows)
            for k in range(DEG_AHEAD):
                pltpu.async_copy(ones_v, acc_sh.at[dst_v.at[k]], dsem,
                                 add=True)

            @pl.loop(0, q - DEG_AHEAD)
            def _(c):
                pltpu.make_async_copy(ones_v, acc_sh.at[dst_v.at[0]],
                                      dsem).wait()
                pltpu.async_copy(ones_v, acc_sh.at[dst_v.at[c + DEG_AHEAD]],
                                 dsem, add=True)

            @pl.loop(0, DEG_AHEAD)
            def _(c):
                pltpu.make_async_copy(ones_v, acc_sh.at[dst_v.at[0]],
                                      dsem).wait()

        plsc.subcore_barrier()
        pltpu.sync_copy(
            acc_sh.at[pl.ds(sid * rows_per_sub, rows_per_sub)],
            out_hbm.at[cid, pl.ds(sid * rows_per_sub, rows_per_sub)],
        )

    return deg_kernel


def _make_edge_kernel(n_chunks, npad, dim, main_rows):
    rows_per_sub = npad // NS
    q = n_chunks // 4

    @functools.partial(
        pl.kernel,
        out_type=jax.ShapeDtypeStruct((NC, npad, dim), jnp.float32),
        mesh=_mesh(),
        scratch_types=(
            [
                pltpu.VMEM((q, CHUNK), jnp.int32),
                pltpu.VMEM((q, CHUNK), jnp.int32),
            ]
            + [pltpu.VMEM((CHUNK, dim), jnp.float32)] * NBUF
            + [pltpu.VMEM_SHARED((npad, dim), jnp.float32)]
            + [pltpu.SemaphoreType.DMA] * (2 * NBUF)
        ),
    )
    def edge_kernel(idx_hbm, fill_src_hbm, fill_dst_hbm, hp_hbm, zeros_hbm,
                    out_hbm, *scr):
        src_v, dst_v = scr[0], scr[1]
        rows = scr[2:2 + NBUF]
        acc_sh = scr[2 + NBUF]
        gsem = scr[3 + NBUF:3 + 2 * NBUF]
        ssem = scr[3 + 2 * NBUF:3 + 3 * NBUF]
        cid = lax.axis_index("c")
        sid = lax.axis_index("s")
        wid = sid * NC + cid
        pltpu.sync_copy(
            zeros_hbm.at[pl.ds(sid * rows_per_sub, rows_per_sub)],
            acc_sh.at[pl.ds(sid * rows_per_sub, rows_per_sub)],
        )
        plsc.subcore_barrier()

        for h in range(4):
            base = wid * n_chunks + h * q
            _load_idx(idx_hbm, 0, fill_src_hbm, src_v, base, q, main_rows)
            _load_idx(idx_hbm, 1, fill_dst_hbm, dst_v, base, q, main_rows)

            for k in range(NBUF):
                pltpu.async_copy(hp_hbm.at[src_v.at[k]], rows[k], gsem[k])

            @pl.loop(0, q // NBUF)
            def _(i):
                c = i * NBUF
                for k in range(NBUF):
                    pltpu.make_async_copy(
                        hp_hbm.at[src_v.at[c + k]], rows[k], gsem[k]
                    ).wait()
                    pltpu.async_copy(
                        rows[k], acc_sh.at[dst_v.at[c + k]], ssem[k], add=True
                    )
                for k in range(NBUF):
                    pltpu.make_async_copy(
                        rows[k], acc_sh.at[dst_v.at[c + k]], ssem[k]
                    ).wait()

                    @pl.when(c + NBUF + k < q)
                    def _():
                        pltpu.async_copy(
                            hp_hbm.at[src_v.at[c + NBUF + k]], rows[k], gsem[k]
                        )

        plsc.subcore_barrier()
        pltpu.sync_copy(
            acc_sh.at[pl.ds(sid * rows_per_sub, rows_per_sub)],
            out_hbm.at[cid, pl.ds(sid * rows_per_sub, rows_per_sub)],
        )

    return edge_kernel


def _mm_body(x_ref, w_ref, out_ref):
    out_ref[...] = jnp.dot(x_ref[...], w_ref[...],
                           preferred_element_type=jnp.float32)


def _scale_body(h0_ref, h1_ref, hraw_ref, out_ref):
    deg = h0_ref[0][:, 0:1] + h1_ref[0][:, 0:1] + 1.0
    out_ref[...] = hraw_ref[...] * lax.rsqrt(deg)


def _final_body(a0_ref, a1_ref, hp_ref, h0_ref, h1_ref, b1_ref, w2_ref,
                b2_ref, out_ref):
    deg = h0_ref[0][:, 0:1] + h1_ref[0][:, 0:1] + 1.0
    dinv = lax.rsqrt(deg)
    tot = (a0_ref[0] + a1_ref[0] + hp_ref[...]) * dinv + b1_ref[...]
    r = jnp.maximum(tot, 0.0)
    z = jnp.dot(r, w2_ref[...], preferred_element_type=jnp.float32)
    z = z + b2_ref[...]
    m = jnp.max(z, axis=1, keepdims=True)
    zm = z - m
    out_ref[...] = zm - jnp.log(jnp.sum(jnp.exp(zm), axis=1, keepdims=True))


def kernel(x, edge_index, W1, b1, W2, b2):
    n, in_dim = x.shape
    hid = W1.shape[1]
    out_dim = W2.shape[1]
    e = edge_index.shape[1]

    npad = ((n + 1 + 127) // 128) * 128
    n_chunks = ((-(-e // (NW * CHUNK)) + 7) // 8) * 8
    q = n_chunks // 4
    assert e % CHUNK == 0 and (e // CHUNK) % q == 0, (
        "edge tail must align to whole staging quarters"
    )
    main_rows = e // CHUNK
    fill_rows = NW * n_chunks - main_rows

    idx3 = edge_index.astype(jnp.int32).reshape(2, main_rows, CHUNK)
    ramp = jnp.arange(fill_rows * CHUNK, dtype=jnp.int32)
    fill_dst = (n + jax.lax.rem(ramp, jnp.int32(npad - n))
                ).reshape(fill_rows, CHUNK)
    fill_src = jax.lax.rem(ramp, jnp.int32(n)).reshape(fill_rows, CHUNK)

    zeros16 = jnp.zeros((npad, 16), jnp.float32)
    ones16 = jnp.ones((CHUNK, 16), jnp.float32)
    zerosd = jnp.zeros((npad, hid), jnp.float32)

    hist2 = _make_deg_kernel(n_chunks, npad, main_rows)(
        idx3, fill_dst, zeros16, ones16)

    blk1 = 2000
    h_raw = pl.pallas_call(
        _mm_body,
        grid=(n // blk1,),
        in_specs=[
            pl.BlockSpec((blk1, in_dim), lambda i: (i, 0)),
            pl.BlockSpec((in_dim, hid), lambda i: (0, 0)),
        ],
        out_specs=pl.BlockSpec((blk1, hid), lambda i: (i, 0)),
        out_shape=jax.ShapeDtypeStruct((n, hid), jnp.float32),
    )(x, W1)

    hp = pl.pallas_call(
        _scale_body,
        grid=(n // blk1,),
        in_specs=[
            pl.BlockSpec((1, blk1, 16), lambda i: (0, i, 0)),
            pl.BlockSpec((1, blk1, 16), lambda i: (1, i, 0)),
            pl.BlockSpec((blk1, hid), lambda i: (i, 0)),
        ],
        out_specs=pl.BlockSpec((blk1, hid), lambda i: (i, 0)),
        out_shape=jax.ShapeDtypeStruct((n, hid), jnp.float32),
    )(hist2, hist2, h_raw)

    acc2 = _make_edge_kernel(n_chunks, npad, hid, main_rows)(
        idx3, fill_src, fill_dst, hp, zerosd)

    blk2 = 2000
    out = pl.pallas_call(
        _final_body,
        grid=(n // blk2,),
        in_specs=[
            pl.BlockSpec((1, blk2, hid), lambda i: (0, i, 0)),
            pl.BlockSpec((1, blk2, hid), lambda i: (1, i, 0)),
            pl.BlockSpec((blk2, hid), lambda i: (i, 0)),
            pl.BlockSpec((1, blk2, 16), lambda i: (0, i, 0)),
            pl.BlockSpec((1, blk2, 16), lambda i: (1, i, 0)),
            pl.BlockSpec((1, hid), lambda i: (0, 0)),
            pl.BlockSpec((hid, out_dim), lambda i: (0, 0)),
            pl.BlockSpec((1, out_dim), lambda i: (0, 0)),
        ],
        out_specs=pl.BlockSpec((blk2, out_dim), lambda i: (i, 0)),
        out_shape=jax.ShapeDtypeStruct((n, out_dim), jnp.float32),
    )(acc2, acc2, hp, hist2, hist2,
      b1.reshape(1, hid), W2, b2.reshape(1, out_dim))

    return out


# speedup vs baseline: 39.6272x; 1.0178x over previous
import functools

import jax
import jax.numpy as jnp
from jax import lax
from jax.experimental import pallas as pl
from jax.experimental.pallas import tpu as pltpu
from jax.experimental.pallas import tpu_sc as plsc

NC = 2
NS = 16
NW = NC * NS
CHUNK = 64
NBUF = 4
DEG_AHEAD = 4


def _mesh():
    return plsc.VectorSubcoreMesh(
        core_axis_name="c", subcore_axis_name="s", num_cores=NC, num_subcores=NS
    )


def _load_idx(idx_hbm, j, fill_hbm, buf, start, q, main_rows):
    is_fill = start >= main_rows

    @pl.when(is_fill)
    def _():
        pltpu.sync_copy(fill_hbm.at[pl.ds(start - main_rows, q)], buf)

    @pl.when(jnp.logical_not(is_fill))
    def _():
        pltpu.sync_copy(idx_hbm.at[j, pl.ds(start, q)], buf)


def _make_deg_kernel(n_chunks, npad, main_rows):
    rows_per_sub = npad // NS
    q = n_chunks // 4

    @functools.partial(
        pl.kernel,
        out_type=jax.ShapeDtypeStruct((NC, npad, 16), jnp.float32),
        mesh=_mesh(),
        scratch_types=[
            pltpu.VMEM((q, CHUNK), jnp.int32),
            pltpu.VMEM((CHUNK, 16), jnp.float32),
            pltpu.VMEM_SHARED((npad, 16), jnp.float32),
            pltpu.SemaphoreType.DMA,
        ],
        compiler_params=pltpu.CompilerParams(use_tc_tiling_on_sc=False),
    )
    def deg_kernel(idx_hbm, fill_hbm, zeros_hbm, ones_hbm, out_hbm,
                   dst_v, ones_v, acc_sh, dsem):
        cid = lax.axis_index("c")
        sid = lax.axis_index("s")
        wid = sid * NC + cid
        pltpu.sync_copy(
            zeros_hbm.at[pl.ds(sid * rows_per_sub, rows_per_sub)],
            acc_sh.at[pl.ds(sid * rows_per_sub, rows_per_sub)],
        )
        pltpu.sync_copy(ones_hbm, ones_v)
        plsc.subcore_barrier()

        for h in range(4):
            _load_idx(idx_hbm, 1, fill_hbm, dst_v, wid * n_chunks + h * q,
                      q, main_rows)
            for k in range(DEG_AHEAD):
                pltpu.async_copy(ones_v, acc_sh.at[dst_v.at[k]], dsem,
                                 add=True)

            @pl.loop(0, q - DEG_AHEAD)
            def _(c):
                pltpu.make_async_copy(ones_v, acc_sh.at[dst_v.at[0]],
                                      dsem).wait()
                pltpu.async_copy(ones_v, acc_sh.at[dst_v.at[c + DEG_AHEAD]],
                                 dsem, add=True)

            @pl.loop(0, DEG_AHEAD)
            def _(c):
                pltpu.make_async_copy(ones_v, acc_sh.at[dst_v.at[0]],
                                      dsem).wait()

        plsc.subcore_barrier()
        pltpu.sync_copy(
            acc_sh.at[pl.ds(sid * rows_per_sub, rows_per_sub)],
            out_hbm.at[cid, pl.ds(sid * rows_per_sub, rows_per_sub)],
        )

    return deg_kernel


def _make_edge_kernel(n_chunks, npad, dim, main_rows):
    rows_per_sub = npad // NS
    q = n_chunks // 4

    @functools.partial(
        pl.kernel,
        out_type=jax.ShapeDtypeStruct((NC, npad, dim), jnp.float32),
        mesh=_mesh(),
        scratch_types=(
            [
                pltpu.VMEM((q, CHUNK), jnp.int32),
                pltpu.VMEM((q, CHUNK), jnp.int32),
            ]
            + [pltpu.VMEM((CHUNK, dim), jnp.float32)] * NBUF
            + [pltpu.VMEM_SHARED((npad, dim), jnp.float32)]
            + [pltpu.SemaphoreType.DMA] * (2 * NBUF)
        ),
    )
    def edge_kernel(idx_hbm, fill_src_hbm, fill_dst_hbm, hp_hbm, zeros_hbm,
                    out_hbm, *scr):
        src_v, dst_v = scr[0], scr[1]
        rows = scr[2:2 + NBUF]
        acc_sh = scr[2 + NBUF]
        gsem = scr[3 + NBUF:3 + 2 * NBUF]
        ssem = scr[3 + 2 * NBUF:3 + 3 * NBUF]
        cid = lax.axis_index("c")
        sid = lax.axis_index("s")
        wid = sid * NC + cid
        zdesc = pltpu.async_copy(
            zeros_hbm.at[pl.ds(sid * rows_per_sub, rows_per_sub)],
            acc_sh.at[pl.ds(sid * rows_per_sub, rows_per_sub)],
            gsem[0],
        )
        _load_idx(idx_hbm, 0, fill_src_hbm, src_v, wid * n_chunks, q,
                  main_rows)
        _load_idx(idx_hbm, 1, fill_dst_hbm, dst_v, wid * n_chunks, q,
                  main_rows)
        zdesc.wait()
        for k in range(NBUF):
            pltpu.async_copy(hp_hbm.at[src_v.at[k]], rows[k], gsem[k])
        plsc.subcore_barrier()

        for h in range(4):
            base = wid * n_chunks + h * q
            if h > 0:
                _load_idx(idx_hbm, 0, fill_src_hbm, src_v, base, q, main_rows)
                _load_idx(idx_hbm, 1, fill_dst_hbm, dst_v, base, q, main_rows)

                for k in range(NBUF):
                    pltpu.async_copy(hp_hbm.at[src_v.at[k]], rows[k], gsem[k])

            @pl.loop(0, q // NBUF)
            def _(i):
                c = i * NBUF
                for k in range(NBUF):
                    pltpu.make_async_copy(
                        hp_hbm.at[src_v.at[c + k]], rows[k], gsem[k]
                    ).wait()
                    pltpu.async_copy(
                        rows[k], acc_sh.at[dst_v.at[c + k]], ssem[k], add=True
                    )
                for k in range(NBUF):
                    pltpu.make_async_copy(
                        rows[k], acc_sh.at[dst_v.at[c + k]], ssem[k]
                    ).wait()

                    @pl.when(c + NBUF + k < q)
                    def _():
                        pltpu.async_copy(
                            hp_hbm.at[src_v.at[c + NBUF + k]], rows[k], gsem[k]
                        )

        plsc.subcore_barrier()
        pltpu.sync_copy(
            acc_sh.at[pl.ds(sid * rows_per_sub, rows_per_sub)],
            out_hbm.at[cid, pl.ds(sid * rows_per_sub, rows_per_sub)],
        )

    return edge_kernel


def _mm_body(x_ref, w_ref, out_ref):
    out_ref[...] = jnp.dot(x_ref[...], w_ref[...],
                           preferred_element_type=jnp.float32)


def _scale_body(h0_ref, h1_ref, hraw_ref, out_ref):
    deg = h0_ref[0][:, 0:1] + h1_ref[0][:, 0:1] + 1.0
    out_ref[...] = hraw_ref[...] * lax.rsqrt(deg)


def _final_body(a0_ref, a1_ref, hp_ref, h0_ref, h1_ref, b1_ref, w2_ref,
                b2_ref, out_ref):
    deg = h0_ref[0][:, 0:1] + h1_ref[0][:, 0:1] + 1.0
    dinv = lax.rsqrt(deg)
    tot = (a0_ref[0] + a1_ref[0] + hp_ref[...]) * dinv + b1_ref[...]
    r = jnp.maximum(tot, 0.0)
    z = jnp.dot(r, w2_ref[...], preferred_element_type=jnp.float32)
    z = z + b2_ref[...]
    m = jnp.max(z, axis=1, keepdims=True)
    zm = z - m
    out_ref[...] = zm - jnp.log(jnp.sum(jnp.exp(zm), axis=1, keepdims=True))


def kernel(x, edge_index, W1, b1, W2, b2):
    n, in_dim = x.shape
    hid = W1.shape[1]
    out_dim = W2.shape[1]
    e = edge_index.shape[1]

    npad = ((n + 1 + 127) // 128) * 128
    n_chunks = ((-(-e // (NW * CHUNK)) + 7) // 8) * 8
    q = n_chunks // 4
    assert e % CHUNK == 0 and (e // CHUNK) % q == 0, (
        "edge tail must align to whole staging quarters"
    )
    main_rows = e // CHUNK
    fill_rows = NW * n_chunks - main_rows

    idx3 = edge_index.astype(jnp.int32).reshape(2, main_rows, CHUNK)
    ramp = jnp.arange(fill_rows * CHUNK, dtype=jnp.int32)
    fill_dst = (n + jax.lax.rem(ramp, jnp.int32(npad - n))
                ).reshape(fill_rows, CHUNK)
    fill_src = jax.lax.rem(ramp, jnp.int32(n)).reshape(fill_rows, CHUNK)

    zeros16 = jnp.zeros((npad, 16), jnp.float32)
    ones16 = jnp.ones((CHUNK, 16), jnp.float32)
    zerosd = jnp.zeros((npad, hid), jnp.float32)

    hist2 = _make_deg_kernel(n_chunks, npad, main_rows)(
        idx3, fill_dst, zeros16, ones16)

    blk1 = 2000
    h_raw = pl.pallas_call(
        _mm_body,
        grid=(n // blk1,),
        in_specs=[
            pl.BlockSpec((blk1, in_dim), lambda i: (i, 0)),
            pl.BlockSpec((in_dim, hid), lambda i: (0, 0)),
        ],
        out_specs=pl.BlockSpec((blk1, hid), lambda i: (i, 0)),
        out_shape=jax.ShapeDtypeStruct((n, hid), jnp.float32),
    )(x, W1)

    hp = pl.pallas_call(
        _scale_body,
        grid=(n // blk1,),
        in_specs=[
            pl.BlockSpec((1, blk1, 16), lambda i: (0, i, 0)),
            pl.BlockSpec((1, blk1, 16), lambda i: (1, i, 0)),
            pl.BlockSpec((blk1, hid), lambda i: (i, 0)),
        ],
        out_specs=pl.BlockSpec((blk1, hid), lambda i: (i, 0)),
        out_shape=jax.ShapeDtypeStruct((n, hid), jnp.float32),
    )(hist2, hist2, h_raw)

    acc2 = _make_edge_kernel(n_chunks, npad, hid, main_rows)(
        idx3, fill_src, fill_dst, hp, zerosd)

    blk2 = 2000
    out = pl.pallas_call(
        _final_body,
        grid=(n // blk2,),
        in_specs=[
            pl.BlockSpec((1, blk2, hid), lambda i: (0, i, 0)),
            pl.BlockSpec((1, blk2, hid), lambda i: (1, i, 0)),
            pl.BlockSpec((blk2, hid), lambda i: (i, 0)),
            pl.BlockSpec((1, blk2, 16), lambda i: (0, i, 0)),
            pl.BlockSpec((1, blk2, 16), lambda i: (1, i, 0)),
            pl.BlockSpec((1, hid), lambda i: (0, 0)),
            pl.BlockSpec((hid, out_dim), lambda i: (0, 0)),
            pl.BlockSpec((1, out_dim), lambda i: (0, 0)),
        ],
        out_specs=pl.BlockSpec((blk2, out_dim), lambda i: (i, 0)),
        out_shape=jax.ShapeDtypeStruct((n, out_dim), jnp.float32),
    )(acc2, acc2, hp, hist2, hist2,
      b1.reshape(1, hid), W2, b2.reshape(1, out_dim))

    return out
